# baseline (speedup 1.0000x reference)
import functools
import jax
import jax.numpy as jnp
from jax import lax
from jax.experimental import pallas as pl
from jax.experimental.pallas import tpu as pltpu
from jax.experimental.pallas import tpu_sc as plsc

_NC = 2
_NS = 16
_CHUNK = 128


def _sc_gather_segment_sum(features, st3d, Vp, C, cpt):
    NW = _NC * _NS
    rows_per_tile = Vp // _NS

    mesh = plsc.VectorSubcoreMesh(core_axis_name="c", subcore_axis_name="s")

    @functools.partial(
        pl.kernel,
        mesh=mesh,
        out_type=[
            jax.ShapeDtypeStruct((_NC, Vp, C), jnp.float32),
            jax.ShapeDtypeStruct((NW, Vp), jnp.float32),
        ],
        scratch_types=[
            pltpu.VMEM((2, _CHUNK), jnp.int32),
            pltpu.VMEM((_CHUNK, C), jnp.float32),
            pltpu.VMEM((Vp,), jnp.float32),
            pltpu.VMEM_SHARED((Vp, C), jnp.float32),
            pltpu.SemaphoreType.DMA,
        ],
        compiler_params=pltpu.CompilerParams(needs_layout_passes=False),
    )
    def k(feat_hbm, st_hbm, out_hbm, cnt_hbm,
          idx2, buf0, cnt_v, acc_sh, sem0):
        c = lax.axis_index("c")
        s = lax.axis_index("s")
        wid = s * _NC + c
        z16 = jnp.zeros((16,), jnp.float32)
        ones16 = jnp.ones((16,), jnp.float32)
        row0 = wid * cpt

        def zc(i, carry):
            cnt_v[pl.ds(i * 16, 16)] = z16
            return carry

        lax.fori_loop(0, Vp // 16, zc, 0)

        def zr(i, carry):
            for q in range(C // 16):
                buf0[i, pl.ds(q * 16, 16)] = z16
            return carry

        lax.fori_loop(0, _CHUNK, zr, 0)

        base = s * rows_per_tile
        nfull, rem = divmod(rows_per_tile, _CHUNK)
        for q in range(nfull):
            pltpu.sync_copy(buf0.at[:],
                            acc_sh.at[pl.ds(base + q * _CHUNK, _CHUNK)])
        if rem:
            pltpu.sync_copy(buf0.at[pl.ds(0, rem)],
                            acc_sh.at[pl.ds(base + nfull * _CHUNK, rem)])
        plsc.subcore_barrier()

        def body(j, carry):
            pltpu.sync_copy(st_hbm.at[row0 + j], idx2)
            pltpu.async_copy(feat_hbm.at[idx2.at[0]], buf0, sem0).wait()
            pltpu.sync_copy(buf0, acc_sh.at[idx2.at[1]], add=True)
            for q in range(_CHUNK // 16):
                plsc.addupdate_scatter(
                    cnt_v, [idx2[1, pl.ds(q * 16, 16)]], ones16)
            return carry

        lax.fori_loop(0, cpt, body, 0)

        plsc.subcore_barrier()
        tile_rows = pl.ds(base, rows_per_tile)
        pltpu.sync_copy(acc_sh.at[tile_rows], out_hbm.at[c, tile_rows])
        pltpu.sync_copy(cnt_v, cnt_hbm.at[wid])

    return k(features, st3d)


def _tc_combine(features, parts0, parts1, cnts_t, a_t, w2_t, b2d, V, C, OUT):
    R = 400
    assert V % R == 0
    NW = _NC * _NS

    def body(f_ref, p0_ref, p1_ref, c_ref, at_ref, w2t_ref, b_ref, o_ref):
        ssum = p0_ref[...] + p1_ref[...]
        cnt = jnp.sum(c_ref[...], axis=1, keepdims=True)
        y = jnp.dot(f_ref[...], at_ref[...], preferred_element_type=jnp.float32)
        z = jnp.dot(ssum, w2t_ref[...], preferred_element_type=jnp.float32)
        denom = jnp.maximum(cnt, 1.0)
        o_ref[...] = jnp.where(cnt > 0.0, y + z / denom + b_ref[...], 0.0)

    return pl.pallas_call(
        body,
        grid=(V // R,),
        in_specs=[
            pl.BlockSpec((R, C), lambda i: (i, 0)),
            pl.BlockSpec((R, C), lambda i: (i, 0)),
            pl.BlockSpec((R, C), lambda i: (i, 0)),
            pl.BlockSpec((R, NW), lambda i: (i, 0)),
            pl.BlockSpec((C, OUT), lambda i: (0, 0)),
            pl.BlockSpec((C, OUT), lambda i: (0, 0)),
            pl.BlockSpec((1, OUT), lambda i: (0, 0)),
        ],
        out_specs=pl.BlockSpec((R, OUT), lambda i: (i, 0)),
        out_shape=jax.ShapeDtypeStruct((V, OUT), jnp.float32),
    )(features, parts0, parts1, cnts_t, a_t, w2_t, b2d)


def kernel(features, edge_index, W, b):
    V, C = features.shape
    E = edge_index.shape[1]
    OUT = W.shape[0]
    NW = _NC * _NS

    Vp = -(-(V + 1) // (_NS * 8)) * (_NS * 8)
    cpt = -(-E // (NW * _CHUNK * 8)) * 8
    Epad = NW * cpt * _CHUNK

    src = edge_index[0]
    tgt = edge_index[1]
    if Epad > E:
        src = jnp.concatenate([src, jnp.zeros((Epad - E,), jnp.int32)])
        tgt = jnp.concatenate([tgt, jnp.full((Epad - E,), V, jnp.int32)])
    st3d = jnp.stack(
        [src.reshape(NW * cpt, _CHUNK), tgt.reshape(NW * cpt, _CHUNK)],
        axis=1)

    parts, cnts = _sc_gather_segment_sum(features, st3d, Vp, C, cpt)

    a_t = (W[:, :C] - W[:, C:]).T
    w2_t = W[:, C:].T
    b2d = b.reshape(1, OUT)
    return _tc_combine(features, parts[0], parts[1], cnts.T, a_t, w2_t, b2d,
                       V, C, OUT)


# speedup vs baseline: 1.4189x; 1.4189x over previous
import functools
import jax
import jax.numpy as jnp
from jax import lax
from jax.experimental import pallas as pl
from jax.experimental.pallas import tpu as pltpu
from jax.experimental.pallas import tpu_sc as plsc

_NC = 2
_NS = 16
_CHUNK = 128


def _sc_gather_segment_sum(features, src2d, tgt2d, Vp, C, cpt):
    NW = _NC * _NS
    rows_per_tile = Vp // _NS

    mesh = plsc.VectorSubcoreMesh(core_axis_name="c", subcore_axis_name="s")

    @functools.partial(
        pl.kernel,
        mesh=mesh,
        out_type=[
            jax.ShapeDtypeStruct((_NC, Vp, C), jnp.float32),
            jax.ShapeDtypeStruct((NW, Vp), jnp.float32),
        ],
        scratch_types=[
            pltpu.VMEM((_CHUNK,), jnp.int32),
            pltpu.VMEM((_CHUNK,), jnp.int32),
            pltpu.VMEM((_CHUNK, C), jnp.float32),
            pltpu.VMEM((Vp,), jnp.float32),
            pltpu.VMEM_SHARED((Vp, C), jnp.float32),
            pltpu.SemaphoreType.DMA,
        ],
        compiler_params=pltpu.CompilerParams(needs_layout_passes=False),
    )
    def k(feat_hbm, src_hbm, tgt_hbm, out_hbm, cnt_hbm,
          idx_s, idx_t, buf0, cnt_v, acc_sh, sem0):
        c = lax.axis_index("c")
        s = lax.axis_index("s")
        wid = s * _NC + c
        z16 = jnp.zeros((16,), jnp.float32)
        ones16 = jnp.ones((16,), jnp.float32)
        row0 = wid * cpt

        def zc(i, carry):
            cnt_v[pl.ds(i * 16, 16)] = z16
            return carry

        lax.fori_loop(0, Vp // 16, zc, 0)

        def zr(i, carry):
            for q in range(C // 16):
                buf0[i, pl.ds(q * 16, 16)] = z16
            return carry

        lax.fori_loop(0, _CHUNK, zr, 0)

        base = s * rows_per_tile
        nfull, rem = divmod(rows_per_tile, _CHUNK)
        for q in range(nfull):
            pltpu.sync_copy(buf0.at[:],
                            acc_sh.at[pl.ds(base + q * _CHUNK, _CHUNK)])
        if rem:
            pltpu.sync_copy(buf0.at[pl.ds(0, rem)],
                            acc_sh.at[pl.ds(base + nfull * _CHUNK, rem)])
        plsc.subcore_barrier()

        def body(j, carry):
            r = row0 + j
            pltpu.sync_copy(src_hbm.at[r], idx_s)
            pltpu.sync_copy(tgt_hbm.at[r], idx_t)
            pltpu.async_copy(feat_hbm.at[idx_s], buf0, sem0).wait()
            pltpu.sync_copy(buf0, acc_sh.at[idx_t], add=True)
            for q in range(_CHUNK // 16):
                plsc.addupdate_scatter(
                    cnt_v, [idx_t[pl.ds(q * 16, 16)]], ones16)
            return carry

        lax.fori_loop(0, cpt, body, 0)

        plsc.subcore_barrier()
        tile_rows = pl.ds(base, rows_per_tile)
        pltpu.sync_copy(acc_sh.at[tile_rows], out_hbm.at[c, tile_rows])
        pltpu.sync_copy(cnt_v, cnt_hbm.at[wid])

    return k(features, src2d, tgt2d)


def _tc_combine(features, parts0, parts1, cnts_t, a_t, w2_t, b2d, V, C, OUT):
    R = 400
    assert V % R == 0
    NW = _NC * _NS

    def body(f_ref, p0_ref, p1_ref, c_ref, at_ref, w2t_ref, b_ref, o_ref):
        ssum = p0_ref[...] + p1_ref[...]
        cnt = jnp.sum(c_ref[...], axis=1, keepdims=True)
        y = jnp.dot(f_ref[...], at_ref[...], preferred_element_type=jnp.float32)
        z = jnp.dot(ssum, w2t_ref[...], preferred_element_type=jnp.float32)
        denom = jnp.maximum(cnt, 1.0)
        o_ref[...] = jnp.where(cnt > 0.0, y + z / denom + b_ref[...], 0.0)

    return pl.pallas_call(
        body,
        grid=(V // R,),
        in_specs=[
            pl.BlockSpec((R, C), lambda i: (i, 0)),
            pl.BlockSpec((R, C), lambda i: (i, 0)),
            pl.BlockSpec((R, C), lambda i: (i, 0)),
            pl.BlockSpec((R, NW), lambda i: (i, 0)),
            pl.BlockSpec((C, OUT), lambda i: (0, 0)),
            pl.BlockSpec((C, OUT), lambda i: (0, 0)),
            pl.BlockSpec((1, OUT), lambda i: (0, 0)),
        ],
        out_specs=pl.BlockSpec((R, OUT), lambda i: (i, 0)),
        out_shape=jax.ShapeDtypeStruct((V, OUT), jnp.float32),
    )(features, parts0, parts1, cnts_t, a_t, w2_t, b2d)


def kernel(features, edge_index, W, b):
    V, C = features.shape
    E = edge_index.shape[1]
    OUT = W.shape[0]
    NW = _NC * _NS

    Vp = -(-(V + 1) // (_NS * 8)) * (_NS * 8)
    cpt = -(-E // (NW * _CHUNK))
    Epad = NW * cpt * _CHUNK

    src = edge_index[0]
    tgt = edge_index[1]
    if Epad > E:
        src = jnp.concatenate([src, jnp.zeros((Epad - E,), jnp.int32)])
        tgt = jnp.concatenate([tgt, jnp.full((Epad - E,), V, jnp.int32)])
    src2d = src.reshape(NW * cpt, _CHUNK)
    tgt2d = tgt.reshape(NW * cpt, _CHUNK)

    parts, cnts = _sc_gather_segment_sum(features, src2d, tgt2d, Vp, C, cpt)

    a_t = (W[:, :C] - W[:, C:]).T
    w2_t = W[:, C:].T
    b2d = b.reshape(1, OUT)
    return _tc_combine(features, parts[0], parts[1], cnts.T, a_t, w2_t, b2d,
                       V, C, OUT)


# speedup vs baseline: 1.5115x; 1.0653x over previous
import functools
import jax
import jax.numpy as jnp
from jax import lax
from jax.experimental import pallas as pl
from jax.experimental.pallas import tpu as pltpu
from jax.experimental.pallas import tpu_sc as plsc

_NC = 2
_NS = 16
_CHUNK = 128


def _sc_gather_segment_sum(features, src2d, tgt2d, Vp, C, cpt):
    NW = _NC * _NS
    rows_per_tile = Vp // _NS

    mesh = plsc.VectorSubcoreMesh(core_axis_name="c", subcore_axis_name="s")

    @functools.partial(
        pl.kernel,
        mesh=mesh,
        out_type=[
            jax.ShapeDtypeStruct((_NC, Vp, C), jnp.float32),
            jax.ShapeDtypeStruct((NW, Vp), jnp.float32),
        ],
        scratch_types=[
            pltpu.VMEM((_CHUNK,), jnp.int32),
            pltpu.VMEM((_CHUNK,), jnp.int32),
            pltpu.VMEM((_CHUNK, C), jnp.float32),
            pltpu.VMEM((Vp,), jnp.float32),
            pltpu.VMEM_SHARED((Vp, C), jnp.float32),
            pltpu.SemaphoreType.DMA,
            pltpu.SemaphoreType.DMA,
        ],
        compiler_params=pltpu.CompilerParams(needs_layout_passes=False),
    )
    def k(feat_hbm, src_hbm, tgt_hbm, out_hbm, cnt_hbm,
          idx_s, idx_t, buf0, cnt_v, acc_sh, sem0, semi):
        c = lax.axis_index("c")
        s = lax.axis_index("s")
        wid = s * _NC + c
        z16 = jnp.zeros((16,), jnp.float32)
        ones16 = jnp.ones((16,), jnp.float32)
        row0 = wid * cpt

        def zc(i, carry):
            cnt_v[pl.ds(i * 16, 16)] = z16
            return carry

        lax.fori_loop(0, Vp // 16, zc, 0)

        def zr(i, carry):
            for q in range(C // 16):
                buf0[i, pl.ds(q * 16, 16)] = z16
            return carry

        lax.fori_loop(0, _CHUNK, zr, 0)

        base = s * rows_per_tile
        nfull, rem = divmod(rows_per_tile, _CHUNK)
        for q in range(nfull):
            pltpu.sync_copy(buf0.at[:],
                            acc_sh.at[pl.ds(base + q * _CHUNK, _CHUNK)])
        if rem:
            pltpu.sync_copy(buf0.at[pl.ds(0, rem)],
                            acc_sh.at[pl.ds(base + nfull * _CHUNK, rem)])
        plsc.subcore_barrier()

        def body(j, carry):
            r = row0 + j
            h = pltpu.async_copy(src_hbm.at[r], idx_s, semi)
            pltpu.sync_copy(tgt_hbm.at[r], idx_t)
            h.wait()
            pltpu.async_copy(feat_hbm.at[idx_s], buf0, sem0).wait()
            pltpu.sync_copy(buf0, acc_sh.at[idx_t], add=True)
            for q in range(_CHUNK // 16):
                plsc.addupdate_scatter(
                    cnt_v, [idx_t[pl.ds(q * 16, 16)]], ones16)
            return carry

        lax.fori_loop(0, cpt, body, 0)

        plsc.subcore_barrier()
        tile_rows = pl.ds(base, rows_per_tile)
        pltpu.sync_copy(acc_sh.at[tile_rows], out_hbm.at[c, tile_rows])
        pltpu.sync_copy(cnt_v, cnt_hbm.at[wid])

    return k(features, src2d, tgt2d)


def _tc_combine(features, parts0, parts1, cnts_t, a_t, w2_t, b2d, V, C, OUT):
    R = 400
    assert V % R == 0
    NW = _NC * _NS

    def body(f_ref, p0_ref, p1_ref, c_ref, at_ref, w2t_ref, b_ref, o_ref):
        ssum = p0_ref[...] + p1_ref[...]
        cnt = jnp.sum(c_ref[...], axis=1, keepdims=True)
        y = jnp.dot(f_ref[...], at_ref[...], preferred_element_type=jnp.float32)
        z = jnp.dot(ssum, w2t_ref[...], preferred_element_type=jnp.float32)
        denom = jnp.maximum(cnt, 1.0)
        o_ref[...] = jnp.where(cnt > 0.0, y + z / denom + b_ref[...], 0.0)

    return pl.pallas_call(
        body,
        grid=(V // R,),
        in_specs=[
            pl.BlockSpec((R, C), lambda i: (i, 0)),
            pl.BlockSpec((R, C), lambda i: (i, 0)),
            pl.BlockSpec((R, C), lambda i: (i, 0)),
            pl.BlockSpec((R, NW), lambda i: (i, 0)),
            pl.BlockSpec((C, OUT), lambda i: (0, 0)),
            pl.BlockSpec((C, OUT), lambda i: (0, 0)),
            pl.BlockSpec((1, OUT), lambda i: (0, 0)),
        ],
        out_specs=pl.BlockSpec((R, OUT), lambda i: (i, 0)),
        out_shape=jax.ShapeDtypeStruct((V, OUT), jnp.float32),
    )(features, parts0, parts1, cnts_t, a_t, w2_t, b2d)


def kernel(features, edge_index, W, b):
    V, C = features.shape
    E = edge_index.shape[1]
    OUT = W.shape[0]
    NW = _NC * _NS

    Vp = -(-(V + 1) // (_NS * 8)) * (_NS * 8)
    cpt = -(-E // (NW * _CHUNK))
    Epad = NW * cpt * _CHUNK

    src = edge_index[0]
    tgt = edge_index[1]
    if Epad > E:
        src = jnp.concatenate([src, jnp.zeros((Epad - E,), jnp.int32)])
        tgt = jnp.concatenate([tgt, jnp.full((Epad - E,), V, jnp.int32)])
    src2d = src.reshape(NW * cpt, _CHUNK)
    tgt2d = tgt.reshape(NW * cpt, _CHUNK)

    parts, cnts = _sc_gather_segment_sum(features, src2d, tgt2d, Vp, C, cpt)

    a_t = (W[:, :C] - W[:, C:]).T
    w2_t = W[:, C:].T
    b2d = b.reshape(1, OUT)
    return _tc_combine(features, parts[0], parts[1], cnts.T, a_t, w2_t, b2d,
                       V, C, OUT)


# speedup vs baseline: 1.6308x; 1.0789x over previous
import functools
import jax
import jax.numpy as jnp
from jax import lax
from jax.experimental import pallas as pl
from jax.experimental.pallas import tpu as pltpu
from jax.experimental.pallas import tpu_sc as plsc

_NC = 2
_NS = 16
_CHUNK = 128


def _sc_gather_segment_sum(features, src2d, tgt2d, Vp, C, cpt):
    NW = _NC * _NS
    rows_per_tile = Vp // _NS

    mesh = plsc.VectorSubcoreMesh(core_axis_name="c", subcore_axis_name="s")

    @functools.partial(
        pl.kernel,
        mesh=mesh,
        out_type=[
            jax.ShapeDtypeStruct((_NC, Vp, C), jnp.float32),
            jax.ShapeDtypeStruct((NW, Vp), jnp.float32),
        ],
        scratch_types=[
            pltpu.VMEM((_CHUNK,), jnp.int32),
            pltpu.VMEM((_CHUNK,), jnp.int32),
            pltpu.VMEM((_CHUNK,), jnp.int32),
            pltpu.VMEM((_CHUNK,), jnp.int32),
            pltpu.VMEM((_CHUNK, C), jnp.float32),
            pltpu.VMEM((Vp,), jnp.float32),
            pltpu.VMEM_SHARED((Vp, C), jnp.float32),
            pltpu.SemaphoreType.DMA,
            pltpu.SemaphoreType.DMA,
            pltpu.SemaphoreType.DMA,
        ],
        compiler_params=pltpu.CompilerParams(needs_layout_passes=False),
    )
    def k(feat_hbm, src_hbm, tgt_hbm, out_hbm, cnt_hbm,
          idx_s0, idx_t0, idx_s1, idx_t1, buf0, cnt_v, acc_sh,
          sem0, semi0, semi1):
        c = lax.axis_index("c")
        s = lax.axis_index("s")
        wid = s * _NC + c
        z16 = jnp.zeros((16,), jnp.float32)
        ones16 = jnp.ones((16,), jnp.float32)
        row0 = wid * cpt

        def zc(i, carry):
            cnt_v[pl.ds(i * 16, 16)] = z16
            return carry

        lax.fori_loop(0, Vp // 16, zc, 0)

        def zr(i, carry):
            for q in range(C // 16):
                buf0[i, pl.ds(q * 16, 16)] = z16
            return carry

        lax.fori_loop(0, _CHUNK, zr, 0)

        base = s * rows_per_tile
        nfull, rem = divmod(rows_per_tile, _CHUNK)
        for q in range(nfull):
            pltpu.sync_copy(buf0.at[:],
                            acc_sh.at[pl.ds(base + q * _CHUNK, _CHUNK)])
        if rem:
            pltpu.sync_copy(buf0.at[pl.ds(0, rem)],
                            acc_sh.at[pl.ds(base + nfull * _CHUNK, rem)])
        plsc.subcore_barrier()

        def prefetch(r, idx_s, idx_t, semi):
            pltpu.async_copy(src_hbm.at[r], idx_s, semi)
            pltpu.async_copy(tgt_hbm.at[r], idx_t, semi)

        def drain(idx_s, idx_t, semi):
            pltpu.make_async_copy(src_hbm.at[0], idx_s, semi).wait()
            pltpu.make_async_copy(tgt_hbm.at[0], idx_t, semi).wait()

        def chunk(j, idx_s, idx_t, semi):
            drain(idx_s, idx_t, semi)
            pltpu.async_copy(feat_hbm.at[idx_s], buf0, sem0).wait()
            pltpu.sync_copy(buf0, acc_sh.at[idx_t], add=True)
            for q in range(_CHUNK // 16):
                plsc.addupdate_scatter(
                    cnt_v, [idx_t[pl.ds(q * 16, 16)]], ones16)

            @pl.when(j + 2 < cpt)
            def _():
                prefetch(row0 + j + 2, idx_s, idx_t, semi)

        prefetch(row0, idx_s0, idx_t0, semi0)
        prefetch(row0 + 1, idx_s1, idx_t1, semi1)

        def body(i, carry):
            chunk(2 * i, idx_s0, idx_t0, semi0)
            chunk(2 * i + 1, idx_s1, idx_t1, semi1)
            return carry

        lax.fori_loop(0, cpt // 2, body, 0)
        if cpt % 2 == 1:
            chunk(cpt - 1, idx_s0, idx_t0, semi0)

        plsc.subcore_barrier()
        tile_rows = pl.ds(base, rows_per_tile)
        pltpu.sync_copy(acc_sh.at[tile_rows], out_hbm.at[c, tile_rows])
        pltpu.sync_copy(cnt_v, cnt_hbm.at[wid])

    return k(features, src2d, tgt2d)


def _tc_combine(features, parts0, parts1, cnts_t, a_t, w2_t, b2d, V, C, OUT):
    R = 400
    assert V % R == 0
    NW = _NC * _NS

    def body(f_ref, p0_ref, p1_ref, c_ref, at_ref, w2t_ref, b_ref, o_ref):
        ssum = p0_ref[...] + p1_ref[...]
        cnt = jnp.sum(c_ref[...], axis=1, keepdims=True)
        y = jnp.dot(f_ref[...], at_ref[...], preferred_element_type=jnp.float32)
        z = jnp.dot(ssum, w2t_ref[...], preferred_element_type=jnp.float32)
        denom = jnp.maximum(cnt, 1.0)
        o_ref[...] = jnp.where(cnt > 0.0, y + z / denom + b_ref[...], 0.0)

    return pl.pallas_call(
        body,
        grid=(V // R,),
        in_specs=[
            pl.BlockSpec((R, C), lambda i: (i, 0)),
            pl.BlockSpec((R, C), lambda i: (i, 0)),
            pl.BlockSpec((R, C), lambda i: (i, 0)),
            pl.BlockSpec((R, NW), lambda i: (i, 0)),
            pl.BlockSpec((C, OUT), lambda i: (0, 0)),
            pl.BlockSpec((C, OUT), lambda i: (0, 0)),
            pl.BlockSpec((1, OUT), lambda i: (0, 0)),
        ],
        out_specs=pl.BlockSpec((R, OUT), lambda i: (i, 0)),
        out_shape=jax.ShapeDtypeStruct((V, OUT), jnp.float32),
    )(features, parts0, parts1, cnts_t, a_t, w2_t, b2d)


def kernel(features, edge_index, W, b):
    V, C = features.shape
    E = edge_index.shape[1]
    OUT = W.shape[0]
    NW = _NC * _NS

    Vp = -(-(V + 1) // (_NS * 8)) * (_NS * 8)
    cpt = -(-E // (NW * _CHUNK))
    Epad = NW * cpt * _CHUNK

    src = edge_index[0]
    tgt = edge_index[1]
    if Epad > E:
        src = jnp.concatenate([src, jnp.zeros((Epad - E,), jnp.int32)])
        tgt = jnp.concatenate([tgt, jnp.full((Epad - E,), V, jnp.int32)])
    src2d = src.reshape(NW * cpt, _CHUNK)
    tgt2d = tgt.reshape(NW * cpt, _CHUNK)

    parts, cnts = _sc_gather_segment_sum(features, src2d, tgt2d, Vp, C, cpt)

    a_t = (W[:, :C] - W[:, C:]).T
    w2_t = W[:, C:].T
    b2d = b.reshape(1, OUT)
    return _tc_combine(features, parts[0], parts[1], cnts.T, a_t, w2_t, b2d,
                       V, C, OUT)


# speedup vs baseline: 1.7338x; 1.0632x over previous
import functools
import jax
import jax.numpy as jnp
from jax import lax
from jax.experimental import pallas as pl
from jax.experimental.pallas import tpu as pltpu
from jax.experimental.pallas import tpu_sc as plsc

_NC = 2
_NS = 16
_CHUNK = 128


def _sc_gather_segment_sum(features, src2d, tgt2d, Vp, C, cpt):
    NW = _NC * _NS
    rows_per_tile = Vp // _NS

    mesh = plsc.VectorSubcoreMesh(core_axis_name="c", subcore_axis_name="s")

    @functools.partial(
        pl.kernel,
        mesh=mesh,
        out_type=[
            jax.ShapeDtypeStruct((_NC, Vp, C), jnp.float32),
            jax.ShapeDtypeStruct((NW, Vp), jnp.float32),
        ],
        scratch_types=[
            pltpu.VMEM((_CHUNK,), jnp.int32),
            pltpu.VMEM((_CHUNK,), jnp.int32),
            pltpu.VMEM((_CHUNK,), jnp.int32),
            pltpu.VMEM((_CHUNK,), jnp.int32),
            pltpu.VMEM((_CHUNK, C), jnp.float32),
            pltpu.VMEM((_CHUNK, C), jnp.float32),
            pltpu.VMEM((Vp,), jnp.float32),
            pltpu.VMEM_SHARED((Vp, C), jnp.float32),
            pltpu.SemaphoreType.DMA,
            pltpu.SemaphoreType.DMA,
            pltpu.SemaphoreType.DMA,
        ],
        compiler_params=pltpu.CompilerParams(needs_layout_passes=False),
    )
    def k(feat_hbm, src_hbm, tgt_hbm, out_hbm, cnt_hbm,
          idx_s0, idx_t0, idx_s1, idx_t1, buf0, buf1, cnt_v, acc_sh,
          sem0, semi0, semi1):
        c = lax.axis_index("c")
        s = lax.axis_index("s")
        wid = s * _NC + c
        z16 = jnp.zeros((16,), jnp.float32)
        ones16 = jnp.ones((16,), jnp.float32)
        row0 = wid * cpt

        def zc(i, carry):
            cnt_v[pl.ds(i * 16, 16)] = z16
            return carry

        lax.fori_loop(0, Vp // 16, zc, 0)

        def zr(i, carry):
            for q in range(C // 16):
                buf0[i, pl.ds(q * 16, 16)] = z16
            return carry

        lax.fori_loop(0, _CHUNK, zr, 0)

        base = s * rows_per_tile
        nfull, rem = divmod(rows_per_tile, _CHUNK)
        for q in range(nfull):
            pltpu.sync_copy(buf0.at[:],
                            acc_sh.at[pl.ds(base + q * _CHUNK, _CHUNK)])
        if rem:
            pltpu.sync_copy(buf0.at[pl.ds(0, rem)],
                            acc_sh.at[pl.ds(base + nfull * _CHUNK, rem)])
        plsc.subcore_barrier()

        def prefetch(r, idx_s, idx_t, semi):
            pltpu.async_copy(src_hbm.at[r], idx_s, semi)
            pltpu.async_copy(tgt_hbm.at[r], idx_t, semi)

        def drain(idx_s, idx_t, semi):
            pltpu.make_async_copy(src_hbm.at[0], idx_s, semi).wait()
            pltpu.make_async_copy(tgt_hbm.at[0], idx_t, semi).wait()

        def scatter_side(j, buf, idx_t, semi_pair):
            idx_s, idx_t_, semi = semi_pair
            pltpu.sync_copy(buf, acc_sh.at[idx_t], add=True)
            for q in range(_CHUNK // 16):
                plsc.addupdate_scatter(
                    cnt_v, [idx_t[pl.ds(q * 16, 16)]], ones16)

            @pl.when(j + 2 < cpt)
            def _():
                prefetch(row0 + j + 2, idx_s, idx_t_, semi)

        prefetch(row0, idx_s0, idx_t0, semi0)
        prefetch(row0 + 1, idx_s1, idx_t1, semi1)

        def body(i, carry):
            j0 = 2 * i
            j1 = 2 * i + 1
            drain(idx_s0, idx_t0, semi0)
            pltpu.async_copy(feat_hbm.at[idx_s0], buf0, sem0).wait()
            drain(idx_s1, idx_t1, semi1)
            h1 = pltpu.async_copy(feat_hbm.at[idx_s1], buf1, sem0)
            scatter_side(j0, buf0, idx_t0, (idx_s0, idx_t0, semi0))
            h1.wait()
            scatter_side(j1, buf1, idx_t1, (idx_s1, idx_t1, semi1))
            return carry

        lax.fori_loop(0, cpt // 2, body, 0)
        if cpt % 2 == 1:
            drain(idx_s0, idx_t0, semi0)
            pltpu.async_copy(feat_hbm.at[idx_s0], buf0, sem0).wait()
            scatter_side(cpt - 1, buf0, idx_t0, (idx_s0, idx_t0, semi0))

        plsc.subcore_barrier()
        tile_rows = pl.ds(base, rows_per_tile)
        pltpu.sync_copy(acc_sh.at[tile_rows], out_hbm.at[c, tile_rows])
        pltpu.sync_copy(cnt_v, cnt_hbm.at[wid])

    return k(features, src2d, tgt2d)


def _tc_combine(features, parts0, parts1, cnts_t, a_t, w2_t, b2d, V, C, OUT):
    R = 400
    assert V % R == 0
    NW = _NC * _NS

    def body(f_ref, p0_ref, p1_ref, c_ref, at_ref, w2t_ref, b_ref, o_ref):
        ssum = p0_ref[...] + p1_ref[...]
        cnt = jnp.sum(c_ref[...], axis=1, keepdims=True)
        y = jnp.dot(f_ref[...], at_ref[...], preferred_element_type=jnp.float32)
        z = jnp.dot(ssum, w2t_ref[...], preferred_element_type=jnp.float32)
        denom = jnp.maximum(cnt, 1.0)
        o_ref[...] = jnp.where(cnt > 0.0, y + z / denom + b_ref[...], 0.0)

    return pl.pallas_call(
        body,
        grid=(V // R,),
        in_specs=[
            pl.BlockSpec((R, C), lambda i: (i, 0)),
            pl.BlockSpec((R, C), lambda i: (i, 0)),
            pl.BlockSpec((R, C), lambda i: (i, 0)),
            pl.BlockSpec((R, NW), lambda i: (i, 0)),
            pl.BlockSpec((C, OUT), lambda i: (0, 0)),
            pl.BlockSpec((C, OUT), lambda i: (0, 0)),
            pl.BlockSpec((1, OUT), lambda i: (0, 0)),
        ],
        out_specs=pl.BlockSpec((R, OUT), lambda i: (i, 0)),
        out_shape=jax.ShapeDtypeStruct((V, OUT), jnp.float32),
    )(features, parts0, parts1, cnts_t, a_t, w2_t, b2d)


def kernel(features, edge_index, W, b):
    V, C = features.shape
    E = edge_index.shape[1]
    OUT = W.shape[0]
    NW = _NC * _NS

    Vp = -(-(V + 1) // (_NS * 8)) * (_NS * 8)
    cpt = -(-E // (NW * _CHUNK))
    Epad = NW * cpt * _CHUNK

    src = edge_index[0]
    tgt = edge_index[1]
    if Epad > E:
        src = jnp.concatenate([src, jnp.zeros((Epad - E,), jnp.int32)])
        tgt = jnp.concatenate([tgt, jnp.full((Epad - E,), V, jnp.int32)])
    src2d = src.reshape(NW * cpt, _CHUNK)
    tgt2d = tgt.reshape(NW * cpt, _CHUNK)

    parts, cnts = _sc_gather_segment_sum(features, src2d, tgt2d, Vp, C, cpt)

    a_t = (W[:, :C] - W[:, C:]).T
    w2_t = W[:, C:].T
    b2d = b.reshape(1, OUT)
    return _tc_combine(features, parts[0], parts[1], cnts.T, a_t, w2_t, b2d,
                       V, C, OUT)
